# baseline (speedup 1.0000x reference)
import functools

import jax
import jax.numpy as jnp
from jax import lax
from jax.experimental import pallas as pl
from jax.experimental.pallas import tpu as pltpu
from jax.experimental.pallas import tpu_sc as plsc

N_CLS_CTX = 4
CTX_DIM = 512
CONTEXT_LEN = 77
PREFIX_LEN = 5
SUFFIX_LEN = CONTEXT_LEN - PREFIX_LEN - N_CLS_CTX
CTX_BEG = PREFIX_LEN
SUF_BEG = PREFIX_LEN + N_CLS_CTX
N_CONST = PREFIX_LEN + SUFFIX_LEN
SLAB = 64
GCH = 16


def _make_sc_stage(num_class: int, b: int, nc: int, b_per_w: int, nw: int):
    n_pieces = b // SLAB
    max_rows = -(-N_CONST // nw)

    @functools.partial(
        pl.kernel,
        mesh=plsc.VectorSubcoreMesh(core_axis_name="c", subcore_axis_name="s"),
        out_type=(
            jax.ShapeDtypeStruct((b, N_CLS_CTX, CTX_DIM), jnp.float32),
            jax.ShapeDtypeStruct((CONTEXT_LEN, b, CTX_DIM), jnp.float32),
        ),
        scratch_types=[
            pltpu.VMEM((b_per_w,), jnp.int32),
            pltpu.VMEM((GCH, N_CLS_CTX, CTX_DIM), jnp.float32),
            pltpu.VMEM((2, SLAB, CTX_DIM), jnp.float32),
            pltpu.SemaphoreType.DMA,
            pltpu.SemaphoreType.DMA,
            pltpu.SemaphoreType.DMA,
        ],
    )
    def stage(table_hbm, idx_hbm, b64_hbm, ctx_hbm, out_hbm,
              idx_v, rows_v, slab_v, gsem, o0, o1):
        osem = (o0, o1)
        wid = lax.axis_index("s") * nc + lax.axis_index("c")
        base = wid * b_per_w
        pltpu.sync_copy(idx_hbm.at[pl.ds(base, b_per_w)], idx_v)
        gcp = pltpu.async_copy(
            table_hbm.at[idx_v.at[pl.ds(0, GCH)]], rows_v, gsem)
        lo = wid * N_CONST // nw
        hi = (wid + 1) * N_CONST // nw

        def drain_slot(s):
            for p in range(n_pieces):
                pltpu.make_async_copy(
                    slab_v.at[s], out_hbm.at[0, pl.ds(p * SLAB, SLAB), :],
                    osem[s]).wait()

        for k in range(max_rows):
            @pl.when(lo + k < hi)
            def _(k=k):
                j = lo + k
                orow = jnp.where(j < PREFIX_LEN, j, j + N_CLS_CTX)
                s = k % 2
                if k >= 2:
                    drain_slot(s)
                sl = slab_v.at[s]
                pltpu.sync_copy(b64_hbm.at[j], sl)
                for p in range(n_pieces):
                    pltpu.async_copy(
                        sl, out_hbm.at[orow, pl.ds(p * SLAB, SLAB), :], osem[s])
        for s in range(2):
            drain_slot(s)
        gcp.wait()
        pltpu.sync_copy(rows_v, ctx_hbm.at[pl.ds(base, GCH)])
        pltpu.async_copy(
            table_hbm.at[idx_v.at[pl.ds(GCH, GCH)]], rows_v, gsem).wait()
        pltpu.sync_copy(rows_v, ctx_hbm.at[pl.ds(base + GCH, GCH)])

    return stage


def _insert_body(ctx_ref, prev_ref, out_ref):
    out_ref[...] = ctx_ref[...]


def _make_tc_insert(b: int, bb: int):
    return pl.pallas_call(
        _insert_body,
        grid=(N_CLS_CTX, b // bb),
        in_specs=[
            pl.BlockSpec((1, bb, CTX_DIM), lambda r, i: (r, i, 0)),
            pl.BlockSpec(memory_space=pl.ANY),
        ],
        out_specs=pl.BlockSpec((1, bb, CTX_DIM), lambda r, i: (CTX_BEG + r, i, 0)),
        out_shape=jax.ShapeDtypeStruct((CONTEXT_LEN, b, CTX_DIM), jnp.float32),
        input_output_aliases={1: 0},
    )


def kernel(label, cls_ctx, token_prefix, token_suffix):
    b = label.shape[0]
    num_class = cls_ctx.shape[0]
    info = plsc.get_sparse_core_info()
    nc, ns = info.num_cores, info.num_subcores
    nw = nc * ns
    assert b % nw == 0 and (b // nw) % GCH == 0 and b % SLAB == 0
    b_per_w = b // nw
    idx = label.astype(jnp.int32)
    cs73 = jnp.concatenate([token_prefix[0], token_suffix[0]], axis=0)
    bcast64 = jnp.broadcast_to(cs73[:, None, :], (N_CONST, SLAB, CTX_DIM))
    ctx, out770 = _make_sc_stage(num_class, b, nc, b_per_w, nw)(
        cls_ctx, idx, bcast64)
    ctx_t = jnp.transpose(ctx, (1, 0, 2))
    out = _make_tc_insert(b, 256)(ctx_t, out770)
    return jnp.transpose(out, (1, 0, 2))


# speedup vs baseline: 1.2527x; 1.2527x over previous
import functools

import jax
import jax.numpy as jnp
from jax import lax
from jax.experimental import pallas as pl
from jax.experimental.pallas import tpu as pltpu
from jax.experimental.pallas import tpu_sc as plsc

N_CLS_CTX = 4
CTX_DIM = 512
CONTEXT_LEN = 77
PREFIX_LEN = 5
SUFFIX_LEN = CONTEXT_LEN - PREFIX_LEN - N_CLS_CTX
CTX_BEG = PREFIX_LEN
SUF_BEG = PREFIX_LEN + N_CLS_CTX


def _make_sc_gather(num_class: int, b: int, nc: int, b_per_w: int):
    @functools.partial(
        pl.kernel,
        mesh=plsc.VectorSubcoreMesh(core_axis_name="c", subcore_axis_name="s"),
        out_type=jax.ShapeDtypeStruct((b, N_CLS_CTX, CTX_DIM), jnp.float32),
        scratch_types=[
            pltpu.VMEM((b_per_w,), jnp.int32),
            pltpu.VMEM((b_per_w, N_CLS_CTX, CTX_DIM), jnp.float32),
            pltpu.SemaphoreType.DMA,
        ],
    )
    def gather(table_hbm, idx_hbm, out_hbm, idx_v, rows_v, sem):
        wid = lax.axis_index("s") * nc + lax.axis_index("c")
        base = wid * b_per_w
        pltpu.sync_copy(idx_hbm.at[pl.ds(base, b_per_w)], idx_v)
        pltpu.async_copy(table_hbm.at[idx_v], rows_v, sem).wait()
        pltpu.sync_copy(rows_v, out_hbm.at[pl.ds(base, b_per_w)])

    return gather


def _const_body(pre_ref, suf_ref, out_ref):
    bb = out_ref.shape[1]
    out_ref[:PREFIX_LEN] = jnp.broadcast_to(
        pre_ref[...], (PREFIX_LEN, bb, CTX_DIM))
    out_ref[SUF_BEG:] = jnp.broadcast_to(
        suf_ref[...], (SUFFIX_LEN, bb, CTX_DIM))


def _make_tc_const(b: int, bb: int):
    return pl.pallas_call(
        _const_body,
        grid=(b // bb,),
        in_specs=[
            pl.BlockSpec((PREFIX_LEN, 1, CTX_DIM), lambda i: (0, 0, 0)),
            pl.BlockSpec((SUFFIX_LEN, 1, CTX_DIM), lambda i: (0, 0, 0)),
        ],
        out_specs=pl.BlockSpec((CONTEXT_LEN, bb, CTX_DIM), lambda i: (0, i, 0)),
        out_shape=jax.ShapeDtypeStruct((CONTEXT_LEN, b, CTX_DIM), jnp.float32),
    )


def _insert_body(ctx_ref, prev_ref, out_ref):
    out_ref[...] = ctx_ref[...]


def _make_tc_insert(b: int, bb: int):
    return pl.pallas_call(
        _insert_body,
        grid=(N_CLS_CTX, b // bb),
        in_specs=[
            pl.BlockSpec((1, bb, CTX_DIM), lambda r, i: (r, i, 0)),
            pl.BlockSpec(memory_space=pl.ANY),
        ],
        out_specs=pl.BlockSpec((1, bb, CTX_DIM), lambda r, i: (CTX_BEG + r, i, 0)),
        out_shape=jax.ShapeDtypeStruct((CONTEXT_LEN, b, CTX_DIM), jnp.float32),
        input_output_aliases={1: 0},
    )


def kernel(label, cls_ctx, token_prefix, token_suffix):
    b = label.shape[0]
    num_class = cls_ctx.shape[0]
    info = plsc.get_sparse_core_info()
    nc, ns = info.num_cores, info.num_subcores
    nw = nc * ns
    assert b % nw == 0 and (b // nw) % 8 == 0
    b_per_w = b // nw
    idx = label.astype(jnp.int32)
    ctx = _make_sc_gather(num_class, b, nc, b_per_w)(cls_ctx, idx)
    ctx_t = jnp.transpose(ctx, (1, 0, 2))
    pre_t = jnp.transpose(token_prefix, (1, 0, 2))
    suf_t = jnp.transpose(token_suffix, (1, 0, 2))
    out1 = _make_tc_const(b, 32)(pre_t, suf_t)
    out = _make_tc_insert(b, 256)(ctx_t, out1)
    return jnp.transpose(out, (1, 0, 2))


# speedup vs baseline: 1.3283x; 1.0603x over previous
import functools

import jax
import jax.numpy as jnp
from jax import lax
from jax.experimental import pallas as pl
from jax.experimental.pallas import tpu as pltpu
from jax.experimental.pallas import tpu_sc as plsc

N_CLS_CTX = 4
CTX_DIM = 512
CONTEXT_LEN = 77
PREFIX_LEN = 5
SUFFIX_LEN = CONTEXT_LEN - PREFIX_LEN - N_CLS_CTX
CTX_BEG = PREFIX_LEN
SUF_BEG = PREFIX_LEN + N_CLS_CTX


def _make_sc_gather(num_class: int, b: int, nc: int, b_per_w: int):
    @functools.partial(
        pl.kernel,
        mesh=plsc.VectorSubcoreMesh(core_axis_name="c", subcore_axis_name="s"),
        out_type=jax.ShapeDtypeStruct((b, N_CLS_CTX, CTX_DIM), jnp.float32),
        scratch_types=[
            pltpu.VMEM((b_per_w,), jnp.int32),
            pltpu.VMEM((b_per_w, N_CLS_CTX, CTX_DIM), jnp.float32),
            pltpu.SemaphoreType.DMA,
        ],
    )
    def gather(table_hbm, idx_hbm, out_hbm, idx_v, rows_v, sem):
        wid = lax.axis_index("s") * nc + lax.axis_index("c")
        base = wid * b_per_w
        pltpu.sync_copy(idx_hbm.at[pl.ds(base, b_per_w)], idx_v)
        pltpu.async_copy(table_hbm.at[idx_v], rows_v, sem).wait()
        pltpu.sync_copy(rows_v, out_hbm.at[pl.ds(base, b_per_w)])

    return gather


def _assemble_body(ctx_ref, pre_ref, suf_ref, out_ref):
    bb = out_ref.shape[1]
    out_ref[:PREFIX_LEN] = jnp.broadcast_to(
        pre_ref[...], (PREFIX_LEN, bb, CTX_DIM))
    out_ref[CTX_BEG:SUF_BEG] = ctx_ref[...]
    out_ref[SUF_BEG:] = jnp.broadcast_to(
        suf_ref[...], (SUFFIX_LEN, bb, CTX_DIM))


def _make_tc_assemble(b: int, bb: int):
    return pl.pallas_call(
        _assemble_body,
        grid=(b // bb,),
        in_specs=[
            pl.BlockSpec((N_CLS_CTX, bb, CTX_DIM), lambda i: (0, i, 0)),
            pl.BlockSpec((PREFIX_LEN, 1, CTX_DIM), lambda i: (0, 0, 0)),
            pl.BlockSpec((SUFFIX_LEN, 1, CTX_DIM), lambda i: (0, 0, 0)),
        ],
        out_specs=pl.BlockSpec((CONTEXT_LEN, bb, CTX_DIM), lambda i: (0, i, 0)),
        out_shape=jax.ShapeDtypeStruct((CONTEXT_LEN, b, CTX_DIM), jnp.float32),
    )


def kernel(label, cls_ctx, token_prefix, token_suffix):
    b = label.shape[0]
    num_class = cls_ctx.shape[0]
    info = plsc.get_sparse_core_info()
    nc, ns = info.num_cores, info.num_subcores
    nw = nc * ns
    assert b % nw == 0 and (b // nw) % 8 == 0
    b_per_w = b // nw
    idx = label.astype(jnp.int32)
    ctx = _make_sc_gather(num_class, b, nc, b_per_w)(cls_ctx, idx)
    ctx_t = jnp.transpose(ctx, (1, 0, 2))
    pre_t = jnp.transpose(token_prefix, (1, 0, 2))
    suf_t = jnp.transpose(token_suffix, (1, 0, 2))
    out770 = _make_tc_assemble(b, 64)(ctx_t, pre_t, suf_t)
    return jnp.transpose(out770, (1, 0, 2))
